# Initial kernel scaffold; baseline (speedup 1.0000x reference)
import jax
import jax.numpy as jnp
from jax.experimental import pallas as pl


def kernel(x_user, x_item, edge_index_user_item, edge_index_item_user, params):
    raise NotImplementedError("write your pallas kernel here")


# speedup vs baseline: 3.0848x; 3.0848x over previous
import functools
import math

import jax
import jax.numpy as jnp
from jax import lax
from jax.experimental import pallas as pl
from jax.experimental.pallas import tpu as pltpu
from jax.experimental.pallas import tpu_sc as plsc

H = 128
_NC = 2
_NS = 16
_NW = _NC * _NS
_C = 80
_L = 16
_BLK = 1000

_f32 = jnp.float32


def _dense_body(xu, xi, pu0, pu1, pi0, pi1,
                qwu, qbu, qwi, qbi, kwu, kbu, kwi, kbi,
                vwu, vbu, vwi, vbi, wui, wiu,
                xu2, xi2, qs_ui, k_ui, v_ui, qs_iu, k_iu, v_iu):
    scale = _f32(1.0 / math.sqrt(H))
    xu_ = xu[...] + pu0[...] + pu1[...]
    xi_ = xi[...] + pi0[...] + pi1[...]
    xu2[...] = xu_
    xi2[...] = xi_
    qu = jnp.dot(xu_, qwu[...], preferred_element_type=_f32) + qbu[...]
    qs_ui[...] = jnp.dot(qu, wui[...], preferred_element_type=_f32) * scale
    k_ui[...] = jnp.dot(xi_, kwi[...], preferred_element_type=_f32) + kbi[...]
    v_ui[...] = jnp.dot(xu_, vwu[...], preferred_element_type=_f32) + vbu[...]
    qi = jnp.dot(xi_, qwi[...], preferred_element_type=_f32) + qbi[...]
    qs_iu[...] = jnp.dot(qi, wiu[...], preferred_element_type=_f32) * scale
    k_iu[...] = jnp.dot(xu_, kwu[...], preferred_element_type=_f32) + kbu[...]
    v_iu[...] = jnp.dot(xi_, vwi[...], preferred_element_type=_f32) + vbi[...]


def _dense_call(xu, xi, pu0, pu1, pi0, pi1, p):
    n = xu.shape[0]
    bx = pl.BlockSpec((_BLK, H), lambda i: (i, 0))
    bw = pl.BlockSpec((H, H), lambda i: (0, 0))
    bb = pl.BlockSpec((1, H), lambda i: (0, 0))
    out_shape = [jax.ShapeDtypeStruct((n, H), _f32)] * 8
    return pl.pallas_call(
        _dense_body,
        grid=(n // _BLK,),
        in_specs=[bx] * 6 + [bw, bb, bw, bb, bw, bb, bw, bb, bw, bb, bw, bb, bw, bw],
        out_specs=[bx] * 8,
        out_shape=out_shape,
    )(xu, xi, pu0, pu1, pi0, pi1,
      p['q_user_w'], p['q_user_b'].reshape(1, H),
      p['q_item_w'], p['q_item_b'].reshape(1, H),
      p['k_user_w'], p['k_user_b'].reshape(1, H),
      p['k_item_w'], p['k_item_b'].reshape(1, H),
      p['v_user_w'], p['v_user_b'].reshape(1, H),
      p['v_item_w'], p['v_item_b'].reshape(1, H),
      p['w_ui'], p['w_iu'])


def _add_body(xu, xi, pu0, pu1, pi0, pi1, ou, oi):
    ou[...] = xu[...] + pu0[...] + pu1[...]
    oi[...] = xi[...] + pi0[...] + pi1[...]


def _add_call(xu, xi, pu0, pu1, pi0, pi1):
    n = xu.shape[0]
    bx = pl.BlockSpec((_BLK, H), lambda i: (i, 0))
    return pl.pallas_call(
        _add_body,
        grid=(n // _BLK,),
        in_specs=[bx] * 6,
        out_specs=[bx] * 2,
        out_shape=[jax.ShapeDtypeStruct((n, H), _f32)] * 2,
    )(xu, xi, pu0, pu1, pi0, pi1)


def _lanesum_body(s_ref, p_ref, l_ref):
    s = jnp.dot(s_ref[0], p_ref[...], preferred_element_type=_f32)
    l_ref[0] = jnp.where(s >= 0, s, _f32(0.01) * s)


def _lanesum_call(s2, pmat):
    shp = s2.shape
    rblk = 5000
    return pl.pallas_call(
        _lanesum_body,
        grid=(shp[0], shp[1] // rblk),
        in_specs=[pl.BlockSpec((1, rblk, H), lambda t, r: (t, r, 0)),
                  pl.BlockSpec((H, 8), lambda t, r: (0, 0))],
        out_specs=pl.BlockSpec((1, rblk, 8), lambda t, r: (t, r, 0)),
        out_shape=jax.ShapeDtypeStruct((shp[0], shp[1], 8), _f32),
    )(s2, pmat)


def _softmax_body(l_ref, a_ref):
    l = l_ref[0]
    m = jnp.max(l)
    e = jnp.exp(l - m)
    a_ref[0] = e / jnp.sum(e)


def _softmax_call(s_ui, s_iu):
    e_num = s_ui.shape[0] // _L
    s2 = jnp.stack([s_ui.reshape(e_num * _L // H, H),
                    s_iu.reshape(e_num * _L // H, H)])
    pmat = (lax.broadcasted_iota(jnp.int32, (H, 8), 0) // _L ==
            lax.broadcasted_iota(jnp.int32, (H, 8), 1)).astype(_f32)
    lk = _lanesum_call(s2, pmat).reshape(2, e_num // H, H)
    blk = (1, e_num // H, H)
    al2 = pl.pallas_call(
        _softmax_body,
        grid=(2,),
        in_specs=[pl.BlockSpec(blk, lambda t: (t, 0, 0))],
        out_specs=pl.BlockSpec(blk, lambda t: (t, 0, 0)),
        out_shape=jax.ShapeDtypeStruct((2, e_num // H, H), _f32),
    )(lk)
    return al2[0].reshape(e_num), al2[1].reshape(e_num)


@functools.cache
def _make_score_kernel(E):
    epw = E // _NW
    assert E % _NW == 0 and epw % _C == 0
    nchunk = epw // _C
    mesh = plsc.VectorSubcoreMesh(core_axis_name="c", subcore_axis_name="s")

    @functools.partial(
        pl.kernel,
        out_type=jax.ShapeDtypeStruct((E * _L,), _f32),
        mesh=mesh,
        scratch_types=[
            pltpu.VMEM((_C,), jnp.int32),
            pltpu.VMEM((_C,), jnp.int32),
            pltpu.VMEM((_C, H), _f32),
            pltpu.VMEM((_C, H), _f32),
            pltpu.VMEM((_C * _L,), _f32),
            pltpu.SemaphoreType.DMA,
            pltpu.SemaphoreType.DMA,
        ],
    )
    def score_kernel(qs_hbm, k_hbm, src_hbm, dst_hbm, out_hbm,
                     sidx, didx, qrows, krows, srows, sem_q, sem_k):
        wid = lax.axis_index("s") * _NC + lax.axis_index("c")
        base = wid * epw

        def chunk_body(ci, carry):
            off = base + ci * _C
            pltpu.sync_copy(src_hbm.at[pl.ds(off, _C)], sidx)
            pltpu.sync_copy(dst_hbm.at[pl.ds(off, _C)], didx)
            cq = pltpu.async_copy(qs_hbm.at[sidx], qrows, sem_q)
            ck = pltpu.async_copy(k_hbm.at[didx], krows, sem_k)
            cq.wait()
            ck.wait()

            def edge_body(e, c2):
                acc = qrows[e, pl.ds(0, _L)] * krows[e, pl.ds(0, _L)]
                for j in range(1, H // _L):
                    acc = acc + (qrows[e, pl.ds(j * _L, _L)] *
                                 krows[e, pl.ds(j * _L, _L)])
                srows[pl.ds(pl.multiple_of(e * _L, 8), _L)] = acc
                return c2

            lax.fori_loop(0, _C, edge_body, 0)
            pltpu.sync_copy(
                srows, out_hbm.at[pl.ds(pl.multiple_of(off * _L, 8), _C * _L)])
            return carry

        lax.fori_loop(0, nchunk, chunk_body, 0)

    return score_kernel


@functools.cache
def _make_agg_kernel(E, N):
    epw = E // _NW
    assert E % _NW == 0 and epw % _C == 0
    nchunk = epw // _C
    npad = ((N + 8 * _NS - 1) // (8 * _NS)) * (8 * _NS)
    rps = npad // _NS
    nfull, rem = divmod(rps, _C)
    mesh = plsc.VectorSubcoreMesh(core_axis_name="c", subcore_axis_name="s")

    @functools.partial(
        pl.kernel,
        out_type=jax.ShapeDtypeStruct((_NC * npad, H), _f32),
        mesh=mesh,
        scratch_types=[
            pltpu.VMEM((_C,), jnp.int32),
            pltpu.VMEM((_C,), jnp.int32),
            pltpu.VMEM((_C, H), _f32),
            pltpu.VMEM((_C,), _f32),
            pltpu.VMEM_SHARED((npad, H), _f32),
            pltpu.SemaphoreType.DMA,
        ],
    )
    def agg_kernel(v_hbm, src_hbm, dst_hbm, alpha_hbm,  out_hbm,
                   sidx, didx, rows, abuf, acc, sem):
        cid = lax.axis_index("c")
        sid = lax.axis_index("s")
        wid = sid * _NC + cid
        base = wid * epw

        zero16 = jnp.zeros((_L,), _f32)

        def zrow(e, c):
            for j in range(H // _L):
                rows[e, pl.ds(j * _L, _L)] = zero16
            return c

        lax.fori_loop(0, _C, zrow, 0)
        r0 = pl.multiple_of(sid * rps, 8)
        for t in range(nfull):
            pltpu.sync_copy(rows, acc.at[pl.ds(pl.multiple_of(r0 + t * _C, 8), _C)])
        if rem:
            pltpu.sync_copy(rows.at[pl.ds(0, rem)],
                            acc.at[pl.ds(pl.multiple_of(r0 + nfull * _C, 8), rem)])
        plsc.subcore_barrier()

        def chunk_body(ci, carry):
            off = base + ci * _C
            pltpu.sync_copy(src_hbm.at[pl.ds(off, _C)], sidx)
            pltpu.sync_copy(dst_hbm.at[pl.ds(off, _C)], didx)
            pltpu.sync_copy(alpha_hbm.at[pl.ds(off, _C)], abuf)
            pltpu.async_copy(v_hbm.at[sidx], rows, sem).wait()

            def blk_body(bi, c2):
                al = abuf[pl.ds(bi * _L, _L)]
                for j in range(_L):
                    e = bi * _L + j
                    av = al[j]
                    for f in range(H // _L):
                        sl = pl.ds(f * _L, _L)
                        rows[e, sl] = rows[e, sl] * av
                return c2

            lax.fori_loop(0, _C // _L, blk_body, 0)
            pltpu.sync_copy(rows, acc.at[didx], add=True)
            return carry

        lax.fori_loop(0, nchunk, chunk_body, 0)
        plsc.subcore_barrier()
        pltpu.sync_copy(acc.at[pl.ds(r0, rps)],
                        out_hbm.at[pl.ds(pl.multiple_of(cid * npad + r0, 8), rps)])

    return agg_kernel, npad


def kernel(x_user, x_item, edge_index_user_item, edge_index_item_user, params):
    n = x_user.shape[0]
    e_num = edge_index_user_item.shape[1]
    src_ui = edge_index_user_item[0]
    dst_ui = edge_index_user_item[1]
    src_iu = edge_index_item_user[0]
    dst_iu = edge_index_item_user[1]

    score_k = _make_score_kernel(e_num)
    agg_k, npad = _make_agg_kernel(e_num, n)

    zeros_part = jnp.zeros((n, H), _f32)
    xu, xi = x_user, x_item
    pu0 = pu1 = pi0 = pi1 = zeros_part
    for lname in ('l1', 'l2'):
        p = params[lname]
        (xu2, xi2, qs_ui, k_ui, v_ui,
         qs_iu, k_iu, v_iu) = _dense_call(xu, xi, pu0, pu1, pi0, pi1, p)
        s_ui = score_k(qs_ui, k_ui, src_ui, dst_ui)
        s_iu = score_k(qs_iu, k_iu, src_iu, dst_iu)
        a_ui, a_iu = _softmax_call(s_ui, s_iu)
        part_i = agg_k(v_ui, src_ui, dst_ui, a_ui)
        part_u = agg_k(v_iu, src_iu, dst_iu, a_iu)
        xu, xi = xu2, xi2
        pu0, pu1 = part_u[:n], part_u[npad:npad + n]
        pi0, pi1 = part_i[:n], part_i[npad:npad + n]

    return _add_call(xu, xi, pu0, pu1, pi0, pi1)


# speedup vs baseline: 6.6931x; 2.1697x over previous
import functools
import math

import jax
import jax.numpy as jnp
from jax import lax
from jax.experimental import pallas as pl
from jax.experimental.pallas import tpu as pltpu
from jax.experimental.pallas import tpu_sc as plsc

H = 128
_NC = 2
_NS = 16
_NW = _NC * _NS
_C = 80
_L = 16
_BLK = 1000

_f32 = jnp.float32


def _dense_body(xu, xi, pu0, pu1, pi0, pi1,
                qwu, qbu, qwi, qbi, kwu, kbu, kwi, kbi,
                vwu, vbu, vwi, vbi, wui, wiu,
                xu2, xi2, qs_ui, k_ui, v_ui, qs_iu, k_iu, v_iu):
    scale = _f32(1.0 / math.sqrt(H))
    xu_ = xu[...] + pu0[...] + pu1[...]
    xi_ = xi[...] + pi0[...] + pi1[...]
    xu2[...] = xu_
    xi2[...] = xi_
    qu = jnp.dot(xu_, qwu[...], preferred_element_type=_f32) + qbu[...]
    qs_ui[...] = jnp.dot(qu, wui[...], preferred_element_type=_f32) * scale
    k_ui[...] = jnp.dot(xi_, kwi[...], preferred_element_type=_f32) + kbi[...]
    v_ui[...] = jnp.dot(xu_, vwu[...], preferred_element_type=_f32) + vbu[...]
    qi = jnp.dot(xi_, qwi[...], preferred_element_type=_f32) + qbi[...]
    qs_iu[...] = jnp.dot(qi, wiu[...], preferred_element_type=_f32) * scale
    k_iu[...] = jnp.dot(xu_, kwu[...], preferred_element_type=_f32) + kbu[...]
    v_iu[...] = jnp.dot(xi_, vwi[...], preferred_element_type=_f32) + vbi[...]


def _dense_call(xu, xi, pu0, pu1, pi0, pi1, p):
    n = xu.shape[0]
    bx = pl.BlockSpec((_BLK, H), lambda i: (i, 0))
    bw = pl.BlockSpec((H, H), lambda i: (0, 0))
    bb = pl.BlockSpec((1, H), lambda i: (0, 0))
    out_shape = [jax.ShapeDtypeStruct((n, H), _f32)] * 8
    return pl.pallas_call(
        _dense_body,
        grid=(n // _BLK,),
        in_specs=[bx] * 6 + [bw, bb, bw, bb, bw, bb, bw, bb, bw, bb, bw, bb, bw, bw],
        out_specs=[bx] * 8,
        out_shape=out_shape,
    )(xu, xi, pu0, pu1, pi0, pi1,
      p['q_user_w'], p['q_user_b'].reshape(1, H),
      p['q_item_w'], p['q_item_b'].reshape(1, H),
      p['k_user_w'], p['k_user_b'].reshape(1, H),
      p['k_item_w'], p['k_item_b'].reshape(1, H),
      p['v_user_w'], p['v_user_b'].reshape(1, H),
      p['v_item_w'], p['v_item_b'].reshape(1, H),
      p['w_ui'], p['w_iu'])


def _add_body(xu, xi, pu0, pu1, pi0, pi1, ou, oi):
    ou[...] = xu[...] + pu0[...] + pu1[...]
    oi[...] = xi[...] + pi0[...] + pi1[...]


def _add_call(xu, xi, pu0, pu1, pi0, pi1):
    n = xu.shape[0]
    bx = pl.BlockSpec((_BLK, H), lambda i: (i, 0))
    return pl.pallas_call(
        _add_body,
        grid=(n // _BLK,),
        in_specs=[bx] * 6,
        out_specs=[bx] * 2,
        out_shape=[jax.ShapeDtypeStruct((n, H), _f32)] * 2,
    )(xu, xi, pu0, pu1, pi0, pi1)


def _lanesum_body(s_ref, p_ref, l_ref):
    s = jnp.dot(s_ref[0], p_ref[...], preferred_element_type=_f32)
    l_ref[0] = jnp.where(s >= 0, s, _f32(0.01) * s)


def _lanesum_call(s2, pmat):
    shp = s2.shape
    rblk = 5000
    return pl.pallas_call(
        _lanesum_body,
        grid=(shp[0], shp[1] // rblk),
        in_specs=[pl.BlockSpec((1, rblk, H), lambda t, r: (t, r, 0)),
                  pl.BlockSpec((H, 8), lambda t, r: (0, 0))],
        out_specs=pl.BlockSpec((1, rblk, 8), lambda t, r: (t, r, 0)),
        out_shape=jax.ShapeDtypeStruct((shp[0], shp[1], 8), _f32),
    )(s2, pmat)


def _softmax_body(l_ref, a_ref):
    l = l_ref[0]
    m = jnp.max(l)
    e = jnp.exp(l - m)
    a_ref[0] = e / jnp.sum(e)


def _softmax_call(s_ui, s_iu):
    e_num = s_ui.shape[0] // _L
    s2 = jnp.stack([s_ui.reshape(e_num * _L // H, H),
                    s_iu.reshape(e_num * _L // H, H)])
    pmat = (lax.broadcasted_iota(jnp.int32, (H, 8), 0) // _L ==
            lax.broadcasted_iota(jnp.int32, (H, 8), 1)).astype(_f32)
    lk = _lanesum_call(s2, pmat).reshape(2, e_num // H, H)
    blk = (1, e_num // H, H)
    al2 = pl.pallas_call(
        _softmax_body,
        grid=(2,),
        in_specs=[pl.BlockSpec(blk, lambda t: (t, 0, 0))],
        out_specs=pl.BlockSpec(blk, lambda t: (t, 0, 0)),
        out_shape=jax.ShapeDtypeStruct((2, e_num // H, H), _f32),
    )(lk)
    return al2[0].reshape(e_num), al2[1].reshape(e_num)


@functools.cache
def _make_score_kernel(E):
    epw = E // _NW
    assert E % _NW == 0 and epw % _C == 0
    nchunk = epw // _C
    assert nchunk % 2 == 1 and nchunk >= 3
    nhalf = (nchunk - 1) // 2
    mesh = plsc.VectorSubcoreMesh(core_axis_name="c", subcore_axis_name="s")

    @functools.partial(
        pl.kernel,
        out_type=jax.ShapeDtypeStruct((E * _L,), _f32),
        mesh=mesh,
        scratch_types=[
            pltpu.VMEM((epw,), jnp.int32),
            pltpu.VMEM((epw,), jnp.int32),
            pltpu.VMEM((_C, H), _f32),
            pltpu.VMEM((_C, H), _f32),
            pltpu.VMEM((_C, H), _f32),
            pltpu.VMEM((_C, H), _f32),
            pltpu.VMEM((_C * _L,), _f32),
            pltpu.VMEM((_C * _L,), _f32),
            pltpu.SemaphoreType.DMA,
            pltpu.SemaphoreType.DMA,
            pltpu.SemaphoreType.DMA,
            pltpu.SemaphoreType.DMA,
            pltpu.SemaphoreType.DMA,
            pltpu.SemaphoreType.DMA,
        ],
    )
    def score_kernel(qs_hbm, k_hbm, src_hbm, dst_hbm, out_hbm,
                     sidx_all, didx_all, qrows0, krows0, qrows1, krows1,
                     srows0, srows1, sem_q0, sem_k0, sem_q1, sem_k1,
                     sem_w0, sem_w1):
        wid = lax.axis_index("s") * _NC + lax.axis_index("c")
        base = wid * epw
        qrows = (qrows0, qrows1)
        krows = (krows0, krows1)
        srows = (srows0, srows1)
        sem_q = (sem_q0, sem_q1)
        sem_k = (sem_k0, sem_k1)
        sem_w = (sem_w0, sem_w1)

        pltpu.sync_copy(src_hbm.at[pl.ds(base, epw)], sidx_all)
        pltpu.sync_copy(dst_hbm.at[pl.ds(base, epw)], didx_all)

        def g_pair(ci, b):
            sl = pl.ds(pl.multiple_of(ci * _C, 8), _C)
            return ((qs_hbm.at[sidx_all.at[sl]], qrows[b], sem_q[b]),
                    (k_hbm.at[didx_all.at[sl]], krows[b], sem_k[b]))

        def g_issue(ci, b):
            for args in g_pair(ci, b):
                pltpu.async_copy(*args)

        def g_wait(ci, b):
            for args in g_pair(ci, b):
                pltpu.make_async_copy(*args).wait()

        def w_desc(ci, b):
            off_l = pl.multiple_of((base + ci * _C) * _L, 8)
            return (srows[b], out_hbm.at[pl.ds(off_l, _C * _L)], sem_w[b])

        def compute(ci, b):
            qr, kr, sr = qrows[b], krows[b], srows[b]

            def edge_body(e, c2):
                acc = qr[e, pl.ds(0, _L)] * kr[e, pl.ds(0, _L)]
                for j in range(1, H // _L):
                    acc = acc + (qr[e, pl.ds(j * _L, _L)] *
                                 kr[e, pl.ds(j * _L, _L)])
                sr[pl.ds(pl.multiple_of(e * _L, 8), _L)] = acc
                return c2

            lax.fori_loop(0, _C, edge_body, 0)
            pltpu.async_copy(*w_desc(ci, b))

        g_issue(0, 0)

        def pair_body(ci2, carry):
            ci = ci2 * 2
            g_wait(ci, 0)
            g_issue(ci + 1, 1)

            @pl.when(ci2 >= 1)
            def _():
                pltpu.make_async_copy(*w_desc(ci, 0)).wait()

            compute(ci, 0)
            g_wait(ci + 1, 1)
            g_issue(ci + 2, 0)

            @pl.when(ci2 >= 1)
            def _():
                pltpu.make_async_copy(*w_desc(ci + 1, 1)).wait()

            compute(ci + 1, 1)
            return carry

        lax.fori_loop(0, nhalf, pair_body, 0)

        last = nchunk - 1
        g_wait(last, 0)
        pltpu.make_async_copy(*w_desc(last, 0)).wait()
        compute(last, 0)
        pltpu.make_async_copy(*w_desc(last, 0)).wait()
        pltpu.make_async_copy(*w_desc(last - 1, 1)).wait()

    return score_kernel


@functools.cache
def _make_agg_kernel(E, N):
    epw = E // _NW
    assert E % _NW == 0 and epw % _C == 0
    nchunk = epw // _C
    npad = ((N + 8 * _NS - 1) // (8 * _NS)) * (8 * _NS)
    rps = npad // _NS
    nfull, rem = divmod(rps, _C)
    mesh = plsc.VectorSubcoreMesh(core_axis_name="c", subcore_axis_name="s")

    assert nchunk % 2 == 1 and nchunk >= 3
    nhalf = (nchunk - 1) // 2

    @functools.partial(
        pl.kernel,
        out_type=jax.ShapeDtypeStruct((_NC * npad, H), _f32),
        mesh=mesh,
        scratch_types=[
            pltpu.VMEM((epw,), jnp.int32),
            pltpu.VMEM((epw,), _f32),
            pltpu.VMEM((_C,), jnp.int32),
            pltpu.VMEM((_C,), jnp.int32),
            pltpu.VMEM((_C, H), _f32),
            pltpu.VMEM((_C, H), _f32),
            pltpu.VMEM_SHARED((npad, H), _f32),
            pltpu.SemaphoreType.DMA,
            pltpu.SemaphoreType.DMA,
            pltpu.SemaphoreType.DMA,
            pltpu.SemaphoreType.DMA,
            pltpu.SemaphoreType.DMA,
            pltpu.SemaphoreType.DMA,
        ],
    )
    def agg_kernel(v_hbm, src_hbm, dst_hbm, alpha_hbm, out_hbm,
                   sidx_all, aall, didx0, didx1, rows0, rows1, acc,
                   sem_g0, sem_g1, sem_s0, sem_s1, sem_d0, sem_d1):
        cid = lax.axis_index("c")
        sid = lax.axis_index("s")
        wid = sid * _NC + cid
        base = wid * epw
        rows = (rows0, rows1)
        didx = (didx0, didx1)
        sem_g = (sem_g0, sem_g1)
        sem_s = (sem_s0, sem_s1)
        sem_d = (sem_d0, sem_d1)

        zero16 = jnp.zeros((_L,), _f32)

        def zrow(e, c):
            for j in range(H // _L):
                rows0[e, pl.ds(j * _L, _L)] = zero16
            return c

        lax.fori_loop(0, _C, zrow, 0)
        r0 = pl.multiple_of(sid * rps, 8)
        for t in range(nfull):
            pltpu.sync_copy(rows0, acc.at[pl.ds(pl.multiple_of(r0 + t * _C, 8), _C)])
        if rem:
            pltpu.sync_copy(rows0.at[pl.ds(0, rem)],
                            acc.at[pl.ds(pl.multiple_of(r0 + nfull * _C, 8), rem)])
        plsc.subcore_barrier()

        pltpu.sync_copy(src_hbm.at[pl.ds(base, epw)], sidx_all)
        pltpu.sync_copy(alpha_hbm.at[pl.ds(base, epw)], aall)

        def g_desc(ci, b):
            sl = pl.ds(pl.multiple_of(ci * _C, 8), _C)
            return (v_hbm.at[sidx_all.at[sl]], rows[b], sem_g[b])

        def d_desc(ci, b):
            off = pl.multiple_of(base + ci * _C, 8)
            return (dst_hbm.at[pl.ds(off, _C)], didx[b], sem_d[b])

        def s_desc(b):
            return (rows[b], acc.at[didx[b]], sem_s[b])

        def scale(ci, b):
            rr = rows[b]

            def blk_body(bi, c2):
                al = aall[pl.ds(ci * _C + bi * _L, _L)]
                for j in range(_L):
                    e = bi * _L + j
                    av = al[j]
                    for f in range(H // _L):
                        sl = pl.ds(f * _L, _L)
                        rr[e, sl] = rr[e, sl] * av
                return c2

            lax.fori_loop(0, _C // _L, blk_body, 0)

        def slot(ci, b, wait_next_scatter):
            bn = 1 - b
            pltpu.make_async_copy(*d_desc(ci, b)).wait()
            pltpu.make_async_copy(*g_desc(ci, b)).wait()
            if wait_next_scatter is not None:
                @pl.when(wait_next_scatter)
                def _():
                    pltpu.make_async_copy(*s_desc(bn)).wait()
            else:
                pltpu.make_async_copy(*s_desc(bn)).wait()
            pltpu.async_copy(*d_desc(ci + 1, bn))
            pltpu.async_copy(*g_desc(ci + 1, bn))
            scale(ci, b)
            pltpu.async_copy(*s_desc(b), add=True)

        pltpu.async_copy(*d_desc(0, 0))
        pltpu.async_copy(*g_desc(0, 0))

        def pair_body(ci2, carry):
            ci = ci2 * 2
            slot(ci, 0, ci2 >= 1)
            slot(ci + 1, 1, None)
            return carry

        lax.fori_loop(0, nhalf, pair_body, 0)

        last = nchunk - 1
        pltpu.make_async_copy(*d_desc(last, 0)).wait()
        pltpu.make_async_copy(*g_desc(last, 0)).wait()
        scale(last, 0)
        pltpu.async_copy(*s_desc(0), add=True)
        pltpu.make_async_copy(*s_desc(0)).wait()
        pltpu.make_async_copy(*s_desc(1)).wait()
        plsc.subcore_barrier()
        pltpu.sync_copy(acc.at[pl.ds(r0, rps)],
                        out_hbm.at[pl.ds(pl.multiple_of(cid * npad + r0, 8), rps)])

    return agg_kernel, npad


def kernel(x_user, x_item, edge_index_user_item, edge_index_item_user, params):
    n = x_user.shape[0]
    e_num = edge_index_user_item.shape[1]
    src_ui = edge_index_user_item[0]
    dst_ui = edge_index_user_item[1]
    src_iu = edge_index_item_user[0]
    dst_iu = edge_index_item_user[1]

    score_k = _make_score_kernel(e_num)
    agg_k, npad = _make_agg_kernel(e_num, n)

    zeros_part = jnp.zeros((n, H), _f32)
    xu, xi = x_user, x_item
    pu0 = pu1 = pi0 = pi1 = zeros_part
    for lname in ('l1', 'l2'):
        p = params[lname]
        (xu2, xi2, qs_ui, k_ui, v_ui,
         qs_iu, k_iu, v_iu) = _dense_call(xu, xi, pu0, pu1, pi0, pi1, p)
        s_ui = score_k(qs_ui, k_ui, src_ui, dst_ui)
        s_iu = score_k(qs_iu, k_iu, src_iu, dst_iu)
        a_ui, a_iu = _softmax_call(s_ui, s_iu)
        part_i = agg_k(v_ui, src_ui, dst_ui, a_ui)
        part_u = agg_k(v_iu, src_iu, dst_iu, a_iu)
        xu, xi = xu2, xi2
        pu0, pu1 = part_u[:n], part_u[npad:npad + n]
        pi0, pi1 = part_i[:n], part_i[npad:npad + n]

    return _add_call(xu, xi, pu0, pu1, pi0, pi1)
